# baseline (speedup 1.0000x reference)
import jax
import jax.numpy as jnp
from jax import lax
from jax.experimental import pallas as pl
from jax.experimental.pallas import tpu as pltpu
from jax.experimental.pallas import tpu_sc as plsc

_SEQ = 50
_BATCH = 16384
_D = 64
_NW = 32
_CB_PER_W = _BATCH // 128 // _NW
_NCH = _SEQ * _CB_PER_W


def _emb_body(table_hbm, idx_hbm, out_hbm, idx_v, buf0, buf1, bt0, bt1,
              sem_g, sem_s):
    wid = lax.axis_index("s") * 2 + lax.axis_index("c")
    col0 = pl.multiple_of(wid * (128 * _CB_PER_W), 8)
    bufs = (buf0, buf1)
    bts = (bt0, bt1)

    pltpu.sync_copy(idx_hbm.at[:, pl.ds(col0, 128 * _CB_PER_W)], idx_v)

    lanes = [jax.lax.iota(jnp.int32, 16) + (16 * l0) for l0 in range(8)]

    def fire_gather(i, slot):
        s, cb = i // _CB_PER_W, i % _CB_PER_W
        pltpu.async_copy(table_hbm.at[idx_v.at[s, pl.ds(cb * 128, 128)]],
                         bufs[slot], sem_g.at[slot])

    def wait_gather(slot):
        pltpu.make_async_copy(table_hbm.at[pl.ds(0, 128)], bufs[slot],
                              sem_g.at[slot]).wait()

    def transpose(slot):
        buf, bt = bufs[slot], bts[slot]

        def col(d, carry):
            didx = jnp.full((16,), d, jnp.int32)
            for l0 in range(8):
                v = plsc.load_gather(buf, [lanes[l0], didx])
                bt[d, pl.ds(16 * l0, 16)] = v
            return carry

        lax.fori_loop(0, _D, col, 0)

    def fire_stores(i, slot):
        s, cb = i // _CB_PER_W, i % _CB_PER_W
        c = wid * _CB_PER_W + cb
        for r in range(8):
            pltpu.async_copy(bts[slot].at[pl.ds(8 * r, 8)],
                             out_hbm.at[s, r, c], sem_s.at[slot])

    def wait_stores(i, slot):
        s, cb = i // _CB_PER_W, i % _CB_PER_W
        c = wid * _CB_PER_W + cb
        for r in range(8):
            pltpu.make_async_copy(bts[slot].at[pl.ds(8 * r, 8)],
                                  out_hbm.at[s, r, c], sem_s.at[slot]).wait()

    fire_gather(0, 0)

    def pair(g, carry):
        for b in range(2):
            i = 2 * g + b

            @pl.when(i + 1 < _NCH)
            def _():
                fire_gather(i + 1, 1 - b)

            wait_gather(b)

            @pl.when(i >= 2)
            def _():
                wait_stores(i - 2, b)

            transpose(b)
            fire_stores(i, b)
        return carry

    lax.fori_loop(0, _NCH // 2, pair, 0)
    wait_stores(_NCH - 2, 0)
    wait_stores(_NCH - 1, 1)


def kernel(x, embedding):
    run = pl.kernel(
        _emb_body,
        mesh=plsc.VectorSubcoreMesh(core_axis_name="c", subcore_axis_name="s"),
        out_type=jax.ShapeDtypeStruct((_SEQ, 8, 128, 8, 128), jnp.float32),
        scratch_types=[
            pltpu.VMEM((_SEQ, 128 * _CB_PER_W), jnp.int32),
            pltpu.VMEM((128, _D), jnp.float32),
            pltpu.VMEM((128, _D), jnp.float32),
            pltpu.VMEM((_D, 128), jnp.float32),
            pltpu.VMEM((_D, 128), jnp.float32),
            pltpu.SemaphoreType.DMA((2,)),
            pltpu.SemaphoreType.DMA((2,)),
        ],
        compiler_params=pltpu.CompilerParams(use_tc_tiling_on_sc=False,
                                             needs_layout_passes=False),
    )
    out5 = run(embedding, x.T)
    return out5.transpose(2, 4, 0, 1, 3).reshape(_BATCH, _SEQ, _D)


# speedup vs baseline: 1.4575x; 1.4575x over previous
import jax
import jax.numpy as jnp
from jax import lax
from jax.experimental import pallas as pl
from jax.experimental.pallas import tpu as pltpu
from jax.experimental.pallas import tpu_sc as plsc

_B = 16384 * 50
_D = 64
_NW = 32
_BPW = _B // _NW
_CH = 128
_NCH = _BPW // _CH
_NBUF = 10
_LOOK = _NBUF - 1
_NGRP = _NCH // _NBUF
_IDXROWS = _BPW // 128


def _emb_body(table_hbm, idx_hbm, out_hbm, idx_v, rows_v, sem_g, sem_s):
    wid = lax.axis_index("s") * 2 + lax.axis_index("c")
    base = wid * _BPW

    idx_row0 = pl.multiple_of(wid * _IDXROWS, 8)
    pltpu.sync_copy(idx_hbm.at[pl.ds(idx_row0, _IDXROWS)], idx_v)

    def fire_gather(i, b):
        pltpu.async_copy(table_hbm.at[idx_v.at[i]], rows_v.at[b],
                         sem_g.at[b])

    def wait_gather(b):
        pltpu.make_async_copy(out_hbm.at[pl.ds(0, _CH)], rows_v.at[b],
                              sem_g.at[b]).wait()

    def fire_store(i, b):
        pltpu.async_copy(rows_v.at[b], out_hbm.at[pl.ds(base + i * _CH, _CH)],
                         sem_s.at[b])

    def wait_store(b):
        pltpu.make_async_copy(rows_v.at[b], out_hbm.at[pl.ds(0, _CH)],
                              sem_s.at[b]).wait()

    for j in range(_LOOK):
        fire_gather(j, j)

    def group(g, carry):
        for b in range(_NBUF):
            i = g * _NBUF + b
            j_slot = (b + _LOOK) % _NBUF

            @pl.when(i + _LOOK < _NCH)
            def _():
                @pl.when(i > 0)
                def _():
                    wait_store(j_slot)
                fire_gather(i + _LOOK, j_slot)

            wait_gather(b)
            fire_store(i, b)
        return carry

    lax.fori_loop(0, _NGRP, group, 0)
    for j in range(_LOOK + 1):
        wait_store(j)


def kernel(x, embedding):
    idx = x.reshape(_B // 128, 128)
    run = pl.kernel(
        _emb_body,
        mesh=plsc.VectorSubcoreMesh(core_axis_name="c", subcore_axis_name="s"),
        out_type=jax.ShapeDtypeStruct((_B, _D), jnp.float32),
        scratch_types=[
            pltpu.VMEM((_IDXROWS, 128), jnp.int32),
            pltpu.VMEM((_NBUF, _CH, _D), jnp.float32),
            pltpu.SemaphoreType.DMA((_NBUF,)),
            pltpu.SemaphoreType.DMA((_NBUF,)),
        ],
        compiler_params=pltpu.CompilerParams(use_tc_tiling_on_sc=False),
    )
    out = run(embedding, idx)
    return out.reshape(x.shape + (_D,))
